# baseline (speedup 1.0000x reference)
import functools

import jax
import jax.numpy as jnp
from jax import lax
from jax.experimental import pallas as pl
from jax.experimental.pallas import tpu as pltpu
from jax.experimental.pallas import tpu_sc as plsc

FIELDS = 8
B = 16384
V = 100000
D = 32
ND = 64
P = 128

_NC = 2
_NS = 16
_NW = _NC * _NS
_BPW = B // _NW
_R4 = V * D // 128


def _make_sc_gather():
    mesh = plsc.VectorSubcoreMesh(core_axis_name="c", subcore_axis_name="s")

    @functools.partial(
        pl.kernel,
        mesh=mesh,
        out_type=jax.ShapeDtypeStruct((FIELDS * D, B), jnp.float32),
        scratch_types=[
            pltpu.VMEM((_BPW,), jnp.int32),
            pltpu.VMEM((_BPW,), jnp.int32),
            pltpu.VMEM((_BPW, 128), jnp.float32),
            pltpu.VMEM((D, _BPW), jnp.float32),
            pltpu.SemaphoreType.DMA,
        ],
        compiler_params=pltpu.CompilerParams(needs_layout_passes=False),
    )
    def sc_gather(i0, i1, i2, i3, i4, i5, i6, i7,
                  t0, t1, t2, t3, t4, t5, t6, t7,
                  out, idx_v, idx4_v, rows4_v, et_v, sem):
        wid = lax.axis_index("s") * _NC + lax.axis_index("c")
        base = wid * _BPW
        idxs = (i0, i1, i2, i3, i4, i5, i6, i7)
        tabs = (t0, t1, t2, t3, t4, t5, t6, t7)
        iota = lax.iota(jnp.int32, 16)
        for f in range(FIELDS):
            pltpu.sync_copy(idxs[f].at[pl.ds(base, _BPW)], idx_v)

            def shift_body(j, _):
                v = idx_v[pl.ds(16 * j, 16)]
                idx4_v[pl.ds(16 * j, 16)] = lax.shift_right_logical(v, 2)
                return 0

            lax.fori_loop(0, _BPW // 16, shift_body, 0, unroll=4)
            pltpu.async_copy(tabs[f].at[idx4_v], rows4_v, sem).wait()

            def ext_body(jb, _):
                i0_ = 16 * jb
                vi = idx_v[pl.ds(i0_, 16)]
                cols = lax.mul(jnp.bitwise_and(vi, 3), 32)
                rows = i0_ + iota

                def d_body(d, _):
                    vals = plsc.load_gather(rows4_v, [rows, cols + d])
                    et_v[d, pl.ds(i0_, 16)] = vals
                    return 0

                lax.fori_loop(0, D, d_body, 0, unroll=4)
                return 0

            lax.fori_loop(0, _BPW // 16, ext_body, 0)
            pltpu.sync_copy(et_v, out.at[pl.ds(f * D, D), pl.ds(base, _BPW)])

    return sc_gather


_SC_GATHER = _make_sc_gather()

_BS = 1024


def _tc_body(emb_ref, num_ref, g_ref, be_ref, wn_ref, bn_ref,
             wcat_ref, wnum_ref, bf_ref, out_ref):
    x = num_ref[...]
    mu = jnp.mean(x, axis=-1, keepdims=True)
    var = jnp.mean((x - mu) ** 2, axis=-1, keepdims=True)
    xn = (x - mu) * lax.rsqrt(var + 1e-5) * g_ref[...] + be_ref[...]
    nf = jnp.maximum(
        jnp.dot(xn, wn_ref[...], preferred_element_type=jnp.float32)
        + bn_ref[...], 0.0)
    et = emb_ref[...]
    acc = jnp.dot(nf, wnum_ref[...], preferred_element_type=jnp.float32)
    acc = acc + lax.dot_general(
        et, wcat_ref[...],
        dimension_numbers=(((0,), (0,)), ((), ())),
        preferred_element_type=jnp.float32)
    out_ref[...] = jnp.maximum(acc + bf_ref[...], 0.0)


def kernel(idx_0, idx_1, idx_2, idx_3, idx_4, idx_5, idx_6, idx_7,
           numeric_input,
           table_0, table_1, table_2, table_3, table_4, table_5, table_6,
           table_7, ln_gamma, ln_beta, W_num, b_num, W_final, b_final):
    tabs4 = [t.reshape(_R4, 128) for t in (table_0, table_1, table_2, table_3,
                                           table_4, table_5, table_6, table_7)]
    emb_t = _SC_GATHER(idx_0, idx_1, idx_2, idx_3, idx_4, idx_5, idx_6, idx_7,
                       *tabs4)
    gam = ln_gamma.reshape(1, ND)
    bet = ln_beta.reshape(1, ND)
    bn = b_num.reshape(1, P)
    bf = b_final.reshape(1, P)
    wcat = W_final[:FIELDS * D]
    wnum = W_final[FIELDS * D:]
    out = pl.pallas_call(
        _tc_body,
        grid=(B // _BS,),
        in_specs=[
            pl.BlockSpec((FIELDS * D, _BS), lambda i: (0, i)),
            pl.BlockSpec((_BS, ND), lambda i: (i, 0)),
            pl.BlockSpec((1, ND), lambda i: (0, 0)),
            pl.BlockSpec((1, ND), lambda i: (0, 0)),
            pl.BlockSpec((ND, P), lambda i: (0, 0)),
            pl.BlockSpec((1, P), lambda i: (0, 0)),
            pl.BlockSpec((FIELDS * D, P), lambda i: (0, 0)),
            pl.BlockSpec((P, P), lambda i: (0, 0)),
            pl.BlockSpec((1, P), lambda i: (0, 0)),
        ],
        out_specs=pl.BlockSpec((_BS, P), lambda i: (i, 0)),
        out_shape=jax.ShapeDtypeStruct((B, P), jnp.float32),
    )(emb_t, numeric_input, gam, bet, W_num, bn, wcat, wnum, bf)
    return out
